# Initial kernel scaffold; baseline (speedup 1.0000x reference)
import jax
import jax.numpy as jnp
from jax.experimental import pallas as pl


def kernel(x, enc_w1, enc_b1, enc_w2, enc_b2, enc_w3, enc_b3, quant_w, quant_b, codebook, pq_w, pq_b, dec_w1, dec_b1, dec_w2, dec_b2, dec_w3, dec_b3):
    raise NotImplementedError("write your pallas kernel here")


# speedup vs baseline: 1.0240x; 1.0240x over previous
import jax
import jax.numpy as jnp
import numpy as np
from jax.experimental import pallas as pl
from jax.experimental.pallas import tpu as pltpu


def _conv(x, w, b, stride=1, pad=1):
    y = jax.lax.conv_general_dilated(
        x, w, (stride, stride), [(pad, pad), (pad, pad)],
        dimension_numbers=('NCHW', 'OIHW', 'NCHW'))
    return y + b[None, :, None, None]


def _up2(x):
    return jnp.repeat(jnp.repeat(x, 2, axis=2), 2, axis=3)


_ROWS = 3136
_BLK = 448
_K = 1024
_D = 64


def _vq_mid_kernel(h_ref, qw_ref, qb_ref, cb_ref, cn_ref, pqw_ref, pqb_ref,
                   g_ref, ls_ref):
    i = pl.program_id(0)
    h = h_ref[...]
    zf = jnp.dot(h, qw_ref[...], preferred_element_type=jnp.float32)
    zf = zf + qb_ref[...]
    s = -2.0 * jnp.dot(zf, cb_ref[...].T, preferred_element_type=jnp.float32)
    s = s + cn_ref[...]
    idx = jnp.argmin(s, axis=1)
    oh = (jax.lax.broadcasted_iota(jnp.int32, (_BLK, _K), 1)
          == idx[:, None]).astype(jnp.float32)
    q = jnp.dot(oh, cb_ref[...], preferred_element_type=jnp.float32)
    dq = q - zf
    part = jnp.sum(dq * dq).reshape(1, 1)
    @pl.when(i == 0)
    def _init():
        ls_ref[...] = jnp.zeros_like(part)
    ls_ref[...] += part
    g = jnp.dot(q, pqw_ref[...], preferred_element_type=jnp.float32)
    g_ref[...] = g + pqb_ref[...]


def _vq_middle(h_vec, quant_w, quant_b, codebook, pq_w, pq_b):
    qw = quant_w[:, :, 0, 0].T
    pqw = pq_w[:, :, 0, 0].T
    cn = jnp.sum(codebook * codebook, axis=1)[None, :]
    g_vec, lsum = pl.pallas_call(
        _vq_mid_kernel,
        grid=(_ROWS // _BLK,),
        in_specs=[
            pl.BlockSpec((_BLK, 256), lambda i: (i, 0)),
            pl.BlockSpec((256, _D), lambda i: (0, 0)),
            pl.BlockSpec((1, _D), lambda i: (0, 0)),
            pl.BlockSpec((_K, _D), lambda i: (0, 0)),
            pl.BlockSpec((1, _K), lambda i: (0, 0)),
            pl.BlockSpec((_D, 256), lambda i: (0, 0)),
            pl.BlockSpec((1, 256), lambda i: (0, 0)),
        ],
        out_specs=[
            pl.BlockSpec((_BLK, 256), lambda i: (i, 0)),
            pl.BlockSpec((1, 1), lambda i: (0, 0)),
        ],
        out_shape=[
            jax.ShapeDtypeStruct((_ROWS, 256), jnp.float32),
            jax.ShapeDtypeStruct((1, 1), jnp.float32),
        ],
    )(h_vec, qw, quant_b[None, :], codebook, cn, pqw, pq_b[None, :])
    commit_loss = lsum[0, 0] / (_ROWS * _D)
    return g_vec, commit_loss


def kernel(x, enc_w1, enc_b1, enc_w2, enc_b2, enc_w3, enc_b3, quant_w,
           quant_b, codebook, pq_w, pq_b, dec_w1, dec_b1, dec_w2, dec_b2,
           dec_w3, dec_b3):
    h = jax.nn.relu(_conv(x, enc_w1, enc_b1, stride=2))
    h = jax.nn.relu(_conv(h, enc_w2, enc_b2, stride=2))
    h = _conv(h, enc_w3, enc_b3, stride=2)
    h_vec = jnp.transpose(h, (0, 2, 3, 1)).reshape(_ROWS, 256)

    g_vec, commit_loss = _vq_middle(h_vec, quant_w, quant_b, codebook,
                                    pq_w, pq_b)

    g = jnp.transpose(g_vec.reshape(4, 28, 28, 256), (0, 3, 1, 2))
    g = jax.nn.relu(_conv(_up2(g), dec_w1, dec_b1))
    g = jax.nn.relu(_conv(_up2(g), dec_w2, dec_b2))
    decoded = _conv(_up2(g), dec_w3, dec_b3)
    return (commit_loss, decoded)


# speedup vs baseline: 1.5438x; 1.5077x over previous
import jax
import jax.numpy as jnp
import numpy as np
from jax.experimental import pallas as pl
from jax.experimental.pallas import tpu as pltpu


def _conv(x, w, b, stride=1, pad=1):
    y = jax.lax.conv_general_dilated(
        x, w, (stride, stride), [(pad, pad), (pad, pad)],
        dimension_numbers=('NCHW', 'OIHW', 'NCHW'))
    return y + b[None, :, None, None]


def _up2(x):
    return jnp.repeat(jnp.repeat(x, 2, axis=2), 2, axis=3)


_ROWS = 3136
_BLK = 448
_K = 1024
_D = 64


def _vq_mid_kernel(h_ref, qw_ref, qb_ref, cb_ref, cn_ref, pqw_ref, pqb_ref,
                   g_ref, ls_ref):
    i = pl.program_id(0)
    h = h_ref[...]
    zf = jnp.dot(h, qw_ref[...], preferred_element_type=jnp.float32)
    zf = zf + qb_ref[...]
    s = -2.0 * jnp.dot(zf, cb_ref[...].T, preferred_element_type=jnp.float32)
    s = s + cn_ref[...]
    idx = jnp.argmin(s, axis=1)
    oh = (jax.lax.broadcasted_iota(jnp.int32, (_BLK, _K), 1)
          == idx[:, None]).astype(jnp.float32)
    q = jnp.dot(oh, cb_ref[...], preferred_element_type=jnp.float32)
    dq = q - zf
    part = jnp.sum(dq * dq).reshape(1, 1)
    @pl.when(i == 0)
    def _init():
        ls_ref[...] = jnp.zeros_like(part)
    ls_ref[...] += part
    g = jnp.dot(q, pqw_ref[...], preferred_element_type=jnp.float32)
    g_ref[...] = g + pqb_ref[...]


def _vq_middle(h_vec, quant_w, quant_b, codebook, pq_w, pq_b):
    qw = quant_w[:, :, 0, 0].T
    pqw = pq_w[:, :, 0, 0].T
    cn = jnp.sum(codebook * codebook, axis=1)[None, :]
    g_vec, lsum = pl.pallas_call(
        _vq_mid_kernel,
        grid=(_ROWS // _BLK,),
        in_specs=[
            pl.BlockSpec((_BLK, 256), lambda i: (i, 0)),
            pl.BlockSpec((256, _D), lambda i: (0, 0)),
            pl.BlockSpec((1, _D), lambda i: (0, 0)),
            pl.BlockSpec((_K, _D), lambda i: (0, 0)),
            pl.BlockSpec((1, _K), lambda i: (0, 0)),
            pl.BlockSpec((_D, 256), lambda i: (0, 0)),
            pl.BlockSpec((1, 256), lambda i: (0, 0)),
        ],
        out_specs=[
            pl.BlockSpec((_BLK, 256), lambda i: (i, 0)),
            pl.BlockSpec((1, 1), lambda i: (0, 0)),
        ],
        out_shape=[
            jax.ShapeDtypeStruct((_ROWS, 256), jnp.float32),
            jax.ShapeDtypeStruct((1, 1), jnp.float32),
        ],
    )(h_vec, qw, quant_b[None, :], codebook, cn, pqw, pq_b[None, :])
    commit_loss = lsum[0, 0] / (_ROWS * _D)
    return g_vec, commit_loss


def _phase_weights(dec_w):
    t = jnp.array([[[1, 0, 0], [0, 1, 1]],
                   [[1, 1, 0], [0, 0, 1]]], jnp.float32)
    w2 = jnp.einsum('aup,bvq,oipq->abuvio', t, t, dec_w)
    co, ci = dec_w.shape[0], dec_w.shape[1]
    return w2.reshape(4, 4, ci, co).astype(jnp.bfloat16)


def _make_upconv_kernel(H, Wp, relu, out_dtype, nchunks, pack_phases):
    L = H * Wp
    CH = L // nchunks

    def _k(x_ref, w_ref, b_ref, o_ref):
        for c in range(nchunks):
            accs = []
            for a in (0, 1):
                for b in (0, 1):
                    acc = jnp.zeros((CH, w_ref.shape[3]), jnp.float32)
                    for u in (0, 1):
                        for v in (0, 1):
                            s = (1 + a + u) * Wp + (b + v - 1) + c * CH
                            xs = x_ref[0, pl.ds(s, CH), :]
                            acc += jnp.dot(xs, w_ref[2 * a + b, 2 * u + v],
                                           preferred_element_type=jnp.float32)
                    acc = acc + b_ref[...]
                    if relu:
                        acc = jnp.maximum(acc, 0.0)
                    acc = acc.astype(out_dtype)
                    if pack_phases:
                        accs.append(acc)
                    else:
                        o_ref[0, 2 * a + b, pl.ds(c * CH, CH), :] = acc
            if pack_phases:
                o_ref[0, pl.ds(c * CH, CH), :] = jnp.concatenate(accs, axis=1)

    return _k


def _upconv(g, w, bias, relu, out_dtype):
    n, h, wdim, ci = g.shape
    co = w.shape[0]
    wp = wdim + 2
    w2 = _phase_weights(w)
    gp = jnp.pad(g.astype(jnp.bfloat16), ((0, 0), (2, 2), (1, 1), (0, 0)))
    gp = gp.reshape(n, (h + 4) * wp, ci)
    l = h * wp
    nchunks = 1 if l <= 4096 else 4
    pack = co < 128
    if pack:
        out_specs = pl.BlockSpec((1, l, 4 * co), lambda i: (i, 0, 0))
        out_shape = jax.ShapeDtypeStruct((n, l, 4 * co), out_dtype)
    else:
        out_specs = pl.BlockSpec((1, 4, l, co), lambda i: (i, 0, 0, 0))
        out_shape = jax.ShapeDtypeStruct((n, 4, l, co), out_dtype)
    out = pl.pallas_call(
        _make_upconv_kernel(h, wp, relu, out_dtype, nchunks, pack),
        grid=(n,),
        in_specs=[
            pl.BlockSpec((1, (h + 4) * wp, ci), lambda i: (i, 0, 0)),
            pl.BlockSpec((4, 4, ci, co), lambda i: (0, 0, 0, 0)),
            pl.BlockSpec((1, co), lambda i: (0, 0)),
        ],
        out_specs=out_specs,
        out_shape=out_shape,
    )(gp, w2, bias[None, :].astype(jnp.float32))
    if pack:
        out = out.reshape(n, h, wp, 2, 2, co)[:, :, 1:wdim + 1]
        out = jnp.transpose(out, (0, 1, 3, 2, 4, 5))
    else:
        out = out.reshape(n, 2, 2, h, wp, co)[:, :, :, :, 1:wdim + 1, :]
        out = jnp.transpose(out, (0, 3, 1, 4, 2, 5))
    return out.reshape(n, 2 * h, 2 * wdim, co)


def kernel(x, enc_w1, enc_b1, enc_w2, enc_b2, enc_w3, enc_b3, quant_w,
           quant_b, codebook, pq_w, pq_b, dec_w1, dec_b1, dec_w2, dec_b2,
           dec_w3, dec_b3):
    h = jax.nn.relu(_conv(x, enc_w1, enc_b1, stride=2))
    h = jax.nn.relu(_conv(h, enc_w2, enc_b2, stride=2))
    h = _conv(h, enc_w3, enc_b3, stride=2)
    h_vec = jnp.transpose(h, (0, 2, 3, 1)).reshape(_ROWS, 256)

    z = _conv(h, quant_w, quant_b, stride=1, pad=0)
    z = jnp.transpose(z, (0, 2, 3, 1))
    zf = z.reshape(-1, 64)
    dd = (jnp.sum(zf * zf, axis=1, keepdims=True) - 2.0 * (zf @ codebook.T)
          + jnp.sum(codebook * codebook, axis=1)[None, :])
    idx = jnp.argmin(dd, axis=1)
    q = jnp.take(codebook, idx, axis=0)
    commit_loss = jnp.mean((q - zf) ** 2)
    pqm = pq_w[:, :, 0, 0].T
    g_vec = q @ pqm + pq_b[None, :]

    g = g_vec.reshape(4, 28, 28, 256)
    g = _upconv(g, dec_w1, dec_b1, relu=True, out_dtype=jnp.bfloat16)
    g = _upconv(g, dec_w2, dec_b2, relu=True, out_dtype=jnp.bfloat16)
    g = _upconv(g, dec_w3, dec_b3, relu=False, out_dtype=jnp.float32)
    decoded = jnp.transpose(g, (0, 3, 1, 2))
    return (commit_loss, decoded)
